# baseline (speedup 1.0000x reference)
import functools

import jax
import jax.numpy as jnp
from jax import lax
from jax.experimental import pallas as pl
from jax.experimental.pallas import tpu as pltpu
from jax.experimental.pallas import tpu_sc as plsc

L_SEQ = 16384
WDIM = 64
PDIM = 64
ODIM = WDIM + PDIM

NUM_CORES = 2
NUM_SUBCORES = 16
NW = NUM_CORES * NUM_SUBCORES
B_PER_W = L_SEQ // NW

_mesh = plsc.VectorSubcoreMesh(core_axis_name="c", subcore_axis_name="s")


@functools.partial(
    pl.kernel,
    mesh=_mesh,
    out_type=jax.ShapeDtypeStruct((L_SEQ, ODIM), jnp.float32),
    compiler_params=pltpu.CompilerParams(needs_layout_passes=False),
    scratch_types=[
        pltpu.VMEM((B_PER_W,), jnp.int32),
        pltpu.VMEM((128, WDIM), jnp.float32),
        pltpu.VMEM((128, PDIM), jnp.float32),
        pltpu.VMEM((128, ODIM), jnp.float32),
        pltpu.SemaphoreType.DMA,
        pltpu.SemaphoreType.DMA,
    ],
)
def _embed_sc(idx_hbm, wword_hbm, wpos_hbm, out_hbm,
              idx_v, rows_v, pos_v, buf_v, gsem, psem):
    CH = 128
    wid = lax.axis_index("s") * NUM_CORES + lax.axis_index("c")
    base = wid * B_PER_W
    pltpu.sync_copy(idx_hbm.at[pl.ds(base, B_PER_W)], idx_v)
    lane = lax.iota(jnp.int32, 16)

    for k in range(B_PER_W // CH):
        poscp = pltpu.async_copy(
            wpos_hbm.at[pl.ds(base + k * CH, CH)], pos_v, psem)

        @plsc.parallel_loop(0, CH, unroll=4)
        def _fetch_row(r):
            rk = k * CH + r
            vec = idx_v[pl.ds((rk // 16) * 16, 16)]
            i = jnp.sum(jnp.where(lane == (rk % 16), vec, 0))
            pltpu.async_copy(wword_hbm.at[i], rows_v.at[r], gsem)

        pltpu.make_async_copy(
            wword_hbm.at[pl.ds(0, CH)], rows_v, gsem).wait()
        poscp.wait()

        @plsc.parallel_loop(0, CH, unroll=4)
        def _interleave(r):
            for c in range(WDIM // 16):
                buf_v[r, pl.ds(c * 16, 16)] = rows_v[r, pl.ds(c * 16, 16)]
            for c in range(PDIM // 16):
                buf_v[r, pl.ds(WDIM + c * 16, 16)] = pos_v[r, pl.ds(c * 16, 16)]

        pltpu.sync_copy(buf_v, out_hbm.at[pl.ds(base + k * CH, CH)])


def kernel(input, W_word, W_pos):
    idx = input.astype(jnp.int32)
    return _embed_sc(idx, W_word, W_pos)


# speedup vs baseline: 1.0010x; 1.0010x over previous
import functools

import jax
import jax.numpy as jnp
from jax import lax
from jax.experimental import pallas as pl
from jax.experimental.pallas import tpu as pltpu
from jax.experimental.pallas import tpu_sc as plsc

L_SEQ = 16384
WDIM = 64
PDIM = 64
ODIM = WDIM + PDIM

NUM_CORES = 2
NUM_SUBCORES = 16
NW = NUM_CORES * NUM_SUBCORES
B_PER_W = L_SEQ // NW
CH = 128
NQ = 4

_mesh = plsc.VectorSubcoreMesh(core_axis_name="c", subcore_axis_name="s")


@functools.partial(
    pl.kernel,
    mesh=_mesh,
    out_type=jax.ShapeDtypeStruct((L_SEQ, ODIM), jnp.float32),
    compiler_params=pltpu.CompilerParams(needs_layout_passes=False),
    scratch_types=[
        pltpu.VMEM((B_PER_W,), jnp.int32),
        pltpu.VMEM((CH, WDIM), jnp.float32),
        pltpu.VMEM((CH, WDIM), jnp.float32),
        pltpu.VMEM((CH, PDIM), jnp.float32),
        pltpu.VMEM((CH, PDIM), jnp.float32),
        pltpu.VMEM((CH, ODIM), jnp.float32),
        pltpu.VMEM((CH, ODIM), jnp.float32),
        [pltpu.SemaphoreType.DMA] * (2 * NQ),
        pltpu.SemaphoreType.DMA,
        pltpu.SemaphoreType.DMA,
        pltpu.SemaphoreType.DMA,
    ],
)
def _embed_sc(idx_hbm, wword_hbm, wpos_hbm, out_hbm,
              idx_v, rows0_v, rows1_v, pos0_v, pos1_v, buf0_v, buf1_v,
              gsems, psem0, psem1, wsem):
    rows_b = (rows0_v, rows1_v)
    pos_b = (pos0_v, pos1_v)
    buf_b = (buf0_v, buf1_v)
    psem_b = (psem0, psem1)
    wid = lax.axis_index("s") * NUM_CORES + lax.axis_index("c")
    base = wid * B_PER_W
    pltpu.sync_copy(idx_hbm.at[pl.ds(base, B_PER_W)], idx_v)
    lane = lax.iota(jnp.int32, 16)
    NCHUNK = B_PER_W // CH
    RQ = CH // NQ

    def issue_chunk(k):
        b = k % 2
        pltpu.async_copy(wpos_hbm.at[pl.ds(base + k * CH, CH)],
                         pos_b[b], psem_b[b])
        for q in range(NQ):

            @plsc.parallel_loop(0, RQ, unroll=4)
            def _fetch_row(r):
                rk = k * CH + q * RQ + r
                vec = idx_v[pl.ds((rk // 16) * 16, 16)]
                i = jnp.sum(jnp.where(lane == (rk % 16), vec, 0))
                pltpu.async_copy(wword_hbm.at[i],
                                 rows_b[b].at[q * RQ + r], gsems[b * NQ + q])

    def finish_chunk(k):
        b = k % 2
        if k >= 2:
            pltpu.make_async_copy(
                buf_b[b], out_hbm.at[pl.ds(0, CH)], wsem).wait()
        for q in range(NQ):
            pltpu.make_async_copy(
                wword_hbm.at[pl.ds(0, RQ)],
                rows_b[b].at[pl.ds(q * RQ, RQ)],
                gsems[b * NQ + q]).wait()
        pltpu.make_async_copy(
            wpos_hbm.at[pl.ds(0, CH)], pos_b[b], psem_b[b]).wait()

        @plsc.parallel_loop(0, CH, unroll=4)
        def _interleave(r):
            for c in range(WDIM // 16):
                buf_b[b][r, pl.ds(c * 16, 16)] = (
                    rows_b[b][r, pl.ds(c * 16, 16)])
            for c in range(PDIM // 16):
                buf_b[b][r, pl.ds(WDIM + c * 16, 16)] = (
                    pos_b[b][r, pl.ds(c * 16, 16)])

        pltpu.async_copy(buf_b[b], out_hbm.at[pl.ds(base + k * CH, CH)], wsem)

    issue_chunk(0)
    for k in range(1, NCHUNK):
        issue_chunk(k)
        finish_chunk(k - 1)
    finish_chunk(NCHUNK - 1)
    for _ in range(2):
        pltpu.make_async_copy(
            buf0_v, out_hbm.at[pl.ds(0, CH)], wsem).wait()


def kernel(input, W_word, W_pos):
    idx = input.astype(jnp.int32)
    return _embed_sc(idx, W_word, W_pos)
